# Initial kernel scaffold; baseline (speedup 1.0000x reference)
import jax
import jax.numpy as jnp
from jax.experimental import pallas as pl


def kernel(x, table, W, b):
    raise NotImplementedError("write your pallas kernel here")


# speedup vs baseline: 6.9944x; 6.9944x over previous
import functools

import jax
import jax.numpy as jnp
from jax import lax
from jax.experimental import pallas as pl
from jax.experimental.pallas import tpu as pltpu
from jax.experimental.pallas import tpu_sc as plsc

_info = plsc.get_sparse_core_info()
_NC, _NS, _L = _info.num_cores, _info.num_subcores, _info.num_lanes
_NW = _NC * _NS

_CH = 128


def _sc_gather(x2d, table, n_idx, d, per_w, nch, field_stride):
    mesh = plsc.VectorSubcoreMesh(core_axis_name="c", subcore_axis_name="s")

    @functools.partial(
        pl.kernel,
        mesh=mesh,
        out_type=jax.ShapeDtypeStruct((n_idx, d), jnp.float32),
        compiler_params=pltpu.CompilerParams(use_tc_tiling_on_sc=False),
        scratch_types=[
            pltpu.VMEM((nch, _CH), jnp.int32),
            pltpu.VMEM((_CH, d), jnp.float32),
            pltpu.VMEM((_CH, d), jnp.float32),
            pltpu.SemaphoreType.DMA,
            pltpu.SemaphoreType.DMA,
        ],
    )
    def k(x_hbm, tab_hbm, out_hbm, idx_v, rows_a, rows_b, sem_a, sem_b):
        wid = lax.axis_index("s") * _NC + lax.axis_index("c")
        pltpu.sync_copy(x_hbm.at[wid], idx_v)

        offv = lax.rem(lax.iota(jnp.int32, 16), jnp.int32(2)) * jnp.int32(
            field_stride
        )

        def add_body(j, carry):
            for s in range(_CH // _L):
                sl = idx_v[j, pl.ds(s * _L, _L)]
                idx_v[j, pl.ds(s * _L, _L)] = sl + offv
            return carry

        lax.fori_loop(0, nch, add_body, 0)

        out0 = wid * per_w
        bufs = (rows_a, rows_b)
        sems = (sem_a, sem_b)

        def start_gather(j, b):
            pltpu.async_copy(tab_hbm.at[idx_v.at[j]], bufs[b], sems[b])

        def drain(b):
            pltpu.make_async_copy(tab_hbm.at[idx_v.at[0]], bufs[b], sems[b]).wait()

        start_gather(0, 0)
        start_gather(1, 1)

        def g_body(i, carry):
            j = i * 2
            for bi in range(2):
                c = j + bi
                drain(bi)
                pltpu.sync_copy(
                    bufs[bi], out_hbm.at[pl.ds(out0 + c * _CH, _CH)]
                )

                @pl.when(c + 2 < nch)
                def _():
                    start_gather(c + 2, bi)
            return carry

        lax.fori_loop(0, nch // 2, g_body, 0)

    return k(x2d, table)


def _tc_project(g2, w, b2, blk=2048):
    m, kdim = g2.shape
    d = w.shape[0]

    def body(g_ref, w_ref, b_ref, o_ref):
        o_ref[...] = (
            lax.dot_general(
                g_ref[...],
                w_ref[...],
                (((1,), (1,)), ((), ())),
                preferred_element_type=jnp.float32,
            )
            + b_ref[...]
        )

    return pl.pallas_call(
        body,
        grid=(m // blk,),
        in_specs=[
            pl.BlockSpec((blk, kdim), lambda i: (i, 0)),
            pl.BlockSpec((d, kdim), lambda i: (0, 0)),
            pl.BlockSpec((1, d), lambda i: (0, 0)),
        ],
        out_specs=pl.BlockSpec((blk, d), lambda i: (i, 0)),
        out_shape=jax.ShapeDtypeStruct((m, d), jnp.float32),
    )(g2, w, b2)


def kernel(x, table, W, b):
    bsz, s, g, f = x.shape
    d = table.shape[1]
    n_idx = bsz * s * g * f
    field_stride = table.shape[0] // f

    per_w = n_idx // _NW
    nch = per_w // _CH

    x2d = x.reshape(_NW, nch, _CH)
    gathered = _sc_gather(x2d, table, n_idx, d, per_w, nch, field_stride)
    out = _tc_project(gathered.reshape(n_idx // f, f * d), W, b.reshape(1, d))
    return out.reshape(bsz, s, g, d)
